# baseline (speedup 1.0000x reference)
import jax
import jax.numpy as jnp
from jax.experimental import pallas as pl

_SZ = 4096
_TEMP = 10.0
_R = 32
_LN2 = 0.6931471805599453


def _gumbel_pair_kernel(g_ref, u_ref, o_ref):
    g = g_ref[...]
    u = u_ref[...]
    t = jnp.log2(u + 1e-20)
    v = g * (0.5 / _TEMP) - jnp.log2(-t) * (0.5 * _LN2 / _TEMP)
    v3 = v.reshape(v.shape[0] // 2, 2, 128)
    s = 0.5 + 0.5 * jnp.tanh(v3[:, 0, :] - v3[:, 1, :])
    o_ref[...] = s.reshape(_R, _SZ)


def _native_view(x):
    return x.reshape(_SZ, 32, 128, 2).transpose(0, 1, 3, 2).reshape(64 * _SZ, 128)


def kernel(gen_matrix, uniform_noise):
    gt = _native_view(gen_matrix)
    ut = _native_view(uniform_noise)
    grid = (_SZ // _R,)
    return pl.pallas_call(
        _gumbel_pair_kernel,
        grid=grid,
        in_specs=[
            pl.BlockSpec((64 * _R, 128), lambda i: (i, 0)),
            pl.BlockSpec((64 * _R, 128), lambda i: (i, 0)),
        ],
        out_specs=pl.BlockSpec((_R, _SZ), lambda i: (i, 0)),
        out_shape=jax.ShapeDtypeStruct((_SZ, _SZ), jnp.float32),
    )(gt, ut)


# speedup vs baseline: 1.1837x; 1.1837x over previous
import jax
import jax.numpy as jnp
from jax.experimental import pallas as pl

_SZ = 4096
_TEMP = 10.0
_R = 64
_LN2 = 0.6931471805599453


def _gumbel_pair_kernel(g_ref, u_ref, o_ref):
    g = g_ref[...]
    u = u_ref[...]
    t = jnp.log2(u + 1e-20)
    v = g * (0.5 / _TEMP) - jnp.log2(-t) * (0.5 * _LN2 / _TEMP)
    v3 = v.reshape(v.shape[0] // 2, 2, 128)
    s = 0.5 + 0.5 * jnp.tanh(v3[:, 0, :] - v3[:, 1, :])
    s3 = s.reshape(_R, 32, 128)
    for t in range(32):
        o_ref[:, 128 * t:128 * (t + 1)] = s3[:, t, :]


def _native_view(x):
    return x.reshape(_SZ, 32, 128, 2).transpose(0, 1, 3, 2).reshape(64 * _SZ, 128)


def kernel(gen_matrix, uniform_noise):
    gt = _native_view(gen_matrix)
    ut = _native_view(uniform_noise)
    grid = (_SZ // _R,)
    return pl.pallas_call(
        _gumbel_pair_kernel,
        grid=grid,
        in_specs=[
            pl.BlockSpec((64 * _R, 128), lambda i: (i, 0)),
            pl.BlockSpec((64 * _R, 128), lambda i: (i, 0)),
        ],
        out_specs=pl.BlockSpec((_R, _SZ), lambda i: (i, 0)),
        out_shape=jax.ShapeDtypeStruct((_SZ, _SZ), jnp.float32),
    )(gt, ut)
